# Initial kernel scaffold; baseline (speedup 1.0000x reference)
import jax
import jax.numpy as jnp
from jax.experimental import pallas as pl


def kernel(x, fc_weight, bias):
    raise NotImplementedError("write your pallas kernel here")


# speedup vs baseline: 1.0681x; 1.0681x over previous
import functools

import jax
import jax.numpy as jnp
from jax import lax
from jax.experimental import pallas as pl
from jax.experimental.pallas import tpu as pltpu
from jax.experimental.pallas import tpu_sc as plsc


def _make_sc_kernel(B, F, b_per_w, NC):
    mesh = plsc.VectorSubcoreMesh(core_axis_name="c", subcore_axis_name="s")

    @functools.partial(
        pl.kernel,
        mesh=mesh,
        out_type=jax.ShapeDtypeStruct((B,), jnp.float32),
        scratch_types=[
            pltpu.VMEM((F, b_per_w), jnp.int32),
            pltpu.VMEM((F, b_per_w), jnp.float32),
            pltpu.VMEM((b_per_w,), jnp.float32),
            pltpu.SemaphoreType.DMA,
        ],
    )
    def sc_k(xT_hbm, table_hbm, out_hbm, idx_v, rows_v, out_v, sem):
        wid = lax.axis_index("s") * NC + lax.axis_index("c")
        base = wid * b_per_w
        pltpu.sync_copy(xT_hbm.at[:, pl.ds(base, b_per_w)], idx_v)
        copies = [
            pltpu.async_copy(table_hbm.at[idx_v.at[f]], rows_v.at[f], sem)
            for f in range(F)
        ]
        for c in copies:
            c.wait()
        for c in range(b_per_w // 16):
            sl = pl.ds(c * 16, 16)
            acc = rows_v[0, sl]
            for f in range(1, F):
                acc = acc + rows_v[f, sl]
            out_v[sl] = acc
        pltpu.sync_copy(out_v, out_hbm.at[pl.ds(base, b_per_w)])

    return sc_k


def kernel(x, fc_weight, bias):
    B, F = x.shape
    info = plsc.get_sparse_core_info()
    NC, NS = info.num_cores, info.num_subcores
    NW = NC * NS
    b_per_w = B // NW

    xT = x.astype(jnp.int32).T
    table = fc_weight.reshape(-1)

    sc_k = _make_sc_kernel(B, F, b_per_w, NC)
    out = sc_k(xT, table)
    return out.reshape(B, 1) + bias


# speedup vs baseline: 5.4088x; 5.0637x over previous
import functools

import jax
import jax.numpy as jnp
from jax import lax
from jax.experimental import pallas as pl
from jax.experimental.pallas import tpu as pltpu
from jax.experimental.pallas import tpu_sc as plsc


def _make_sc_kernel(B, F, b_per_w, NC):
    mesh = plsc.VectorSubcoreMesh(core_axis_name="c", subcore_axis_name="s")

    @functools.partial(
        pl.kernel,
        mesh=mesh,
        out_type=jax.ShapeDtypeStruct((B,), jnp.float32),
        scratch_types=[
            pltpu.VMEM((F, b_per_w), jnp.int32),
            pltpu.VMEM((F, b_per_w), jnp.float32),
            pltpu.VMEM((b_per_w,), jnp.float32),
            pltpu.SemaphoreType.DMA,
        ],
    )
    def sc_k(xT_hbm, table_hbm, out_hbm, idx_v, rows_v, out_v, sem):
        wid = lax.axis_index("s") * NC + lax.axis_index("c")
        base = wid * b_per_w
        pltpu.sync_copy(xT_hbm.at[:, pl.ds(base, b_per_w)], idx_v)
        tbl = table_hbm.at[0]
        copies = [
            pltpu.async_copy(tbl.at[idx_v.at[f]], rows_v.at[f], sem)
            for f in range(F)
        ]
        for c in copies:
            c.wait()
        for c in range(b_per_w // 16):
            sl = pl.ds(c * 16, 16)
            acc = rows_v[0, sl]
            for f in range(1, F):
                acc = acc + rows_v[f, sl]
            out_v[sl] = acc
        pltpu.sync_copy(out_v, out_hbm.at[pl.ds(base, b_per_w)])

    return sc_k


def kernel(x, fc_weight, bias):
    B, F = x.shape
    info = plsc.get_sparse_core_info()
    NC, NS = info.num_cores, info.num_subcores
    NW = NC * NS
    b_per_w = B // NW

    xT = x.astype(jnp.int32).T
    table = fc_weight.reshape(1, -1)

    sc_k = _make_sc_kernel(B, F, b_per_w, NC)
    out = sc_k(xT, table)
    return out.reshape(B, 1) + bias


# speedup vs baseline: 5.6613x; 1.0467x over previous
import functools

import jax
import jax.numpy as jnp
from jax import lax
from jax.experimental import pallas as pl
from jax.experimental.pallas import tpu as pltpu
from jax.experimental.pallas import tpu_sc as plsc


def _make_sc_kernel(B, F, b_per_w, NC):
    mesh = plsc.VectorSubcoreMesh(core_axis_name="c", subcore_axis_name="s")

    @functools.partial(
        pl.kernel,
        mesh=mesh,
        out_type=jax.ShapeDtypeStruct((B,), jnp.float32),
        scratch_types=[
            pltpu.VMEM((F, b_per_w), jnp.int32),
            pltpu.VMEM((F, b_per_w), jnp.float32),
            pltpu.VMEM((b_per_w,), jnp.float32),
            pltpu.VMEM((16,), jnp.float32),
            pltpu.SemaphoreType.DMA,
            pltpu.SemaphoreType.DMA,
        ],
    )
    def sc_k(xT_hbm, table_hbm, bias_hbm, out_hbm, idx_v, rows_v, out_v,
             bias_v, sem, bsem):
        wid = lax.axis_index("s") * NC + lax.axis_index("c")
        base = wid * b_per_w
        pltpu.sync_copy(xT_hbm.at[:, pl.ds(base, b_per_w)], idx_v)
        bias_cp = pltpu.async_copy(bias_hbm, bias_v.at[pl.ds(0, 1)], bsem)
        tbl = table_hbm.at[0]
        copies = [
            pltpu.async_copy(tbl.at[idx_v.at[f]], rows_v.at[f], sem)
            for f in range(F)
        ]
        n_ch = b_per_w // 16
        accs = [None] * n_ch
        for f in range(F):
            copies[f].wait()
            for c in range(n_ch):
                v = rows_v[f, pl.ds(c * 16, 16)]
                accs[c] = v if accs[c] is None else accs[c] + v
        bias_cp.wait()
        b = bias_v[...][0]
        for c in range(n_ch):
            out_v[pl.ds(c * 16, 16)] = accs[c] + b
        pltpu.sync_copy(out_v, out_hbm.at[pl.ds(base, b_per_w)])

    return sc_k


def kernel(x, fc_weight, bias):
    B, F = x.shape
    info = plsc.get_sparse_core_info()
    NC, NS = info.num_cores, info.num_subcores
    NW = NC * NS
    b_per_w = B // NW

    xT = x.astype(jnp.int32).T
    table = fc_weight.reshape(1, -1)

    sc_k = _make_sc_kernel(B, F, b_per_w, NC)
    out = sc_k(xT, table, bias)
    return out.reshape(B, 1)


# speedup vs baseline: 5.8193x; 1.0279x over previous
import functools

import jax
import jax.numpy as jnp
from jax import lax
from jax.experimental import pallas as pl
from jax.experimental.pallas import tpu as pltpu
from jax.experimental.pallas import tpu_sc as plsc


def _make_sc_kernel(B, F, b_per_w, NC):
    mesh = plsc.VectorSubcoreMesh(core_axis_name="c", subcore_axis_name="s")

    @functools.partial(
        pl.kernel,
        mesh=mesh,
        out_type=jax.ShapeDtypeStruct((B,), jnp.float32),
        scratch_types=[
            pltpu.VMEM((F, b_per_w), jnp.int32),
            pltpu.VMEM((F, b_per_w), jnp.float32),
            pltpu.VMEM((b_per_w,), jnp.float32),
            pltpu.VMEM((16,), jnp.float32),
            pltpu.SemaphoreType.DMA,
            pltpu.SemaphoreType.DMA,
            pltpu.SemaphoreType.DMA,
        ],
    )
    def sc_k(xT_hbm, table_hbm, bias_hbm, out_hbm, idx_v, rows_v, out_v,
             bias_v, sem, bsem, csem):
        wid = lax.axis_index("s") * NC + lax.axis_index("c")
        base = wid * b_per_w
        half = F // 2
        tbl = table_hbm.at[0]
        pltpu.sync_copy(xT_hbm.at[:, pl.ds(base, b_per_w)], idx_v)
        copies_a = [
            pltpu.async_copy(tbl.at[idx_v.at[f]], rows_v.at[f], sem)
            for f in range(half)
        ]
        bias_cp = pltpu.async_copy(bias_hbm, bias_v.at[pl.ds(0, 1)], csem)
        copies_b = [
            pltpu.async_copy(tbl.at[idx_v.at[f]], rows_v.at[f], bsem)
            for f in range(half, F)
        ]
        n_ch = b_per_w // 16
        accs = [None] * n_ch
        for c in copies_a:
            c.wait()
        for f in range(half):
            for c in range(n_ch):
                v = rows_v[f, pl.ds(c * 16, 16)]
                accs[c] = v if accs[c] is None else accs[c] + v
        bias_cp.wait()
        for c in copies_b:
            c.wait()
        for f in range(half, F):
            for c in range(n_ch):
                accs[c] = accs[c] + rows_v[f, pl.ds(c * 16, 16)]
        b = bias_v[...][0]
        for c in range(n_ch):
            out_v[pl.ds(c * 16, 16)] = accs[c] + b
        pltpu.sync_copy(out_v, out_hbm.at[pl.ds(base, b_per_w)])

    return sc_k


def kernel(x, fc_weight, bias):
    B, F = x.shape
    info = plsc.get_sparse_core_info()
    NC, NS = info.num_cores, info.num_subcores
    NW = NC * NS
    b_per_w = B // NW

    xT = x.astype(jnp.int32).T
    table = fc_weight.reshape(1, -1)

    sc_k = _make_sc_kernel(B, F, b_per_w, NC)
    out = sc_k(xT, table, bias)
    return out.reshape(B, 1)


# speedup vs baseline: 5.9570x; 1.0237x over previous
import functools

import jax
import jax.numpy as jnp
from jax import lax
from jax.experimental import pallas as pl
from jax.experimental.pallas import tpu as pltpu
from jax.experimental.pallas import tpu_sc as plsc


def _make_sc_kernel(B, F, b_per_w, NC):
    mesh = plsc.VectorSubcoreMesh(core_axis_name="c", subcore_axis_name="s")

    @functools.partial(
        pl.kernel,
        mesh=mesh,
        out_type=jax.ShapeDtypeStruct((B,), jnp.float32),
        scratch_types=[
            pltpu.VMEM((F, b_per_w), jnp.int32),
            pltpu.VMEM((F, b_per_w), jnp.float32),
            pltpu.VMEM((b_per_w,), jnp.float32),
            pltpu.VMEM((16,), jnp.float32),
            pltpu.SemaphoreType.DMA,
            pltpu.SemaphoreType.DMA,
            pltpu.SemaphoreType.DMA,
        ],
    )
    def sc_k(xT_hbm, table_hbm, bias_hbm, out_hbm, idx_v, rows_v, out_v,
             bias_v, sem, bsem, csem):
        wid = lax.axis_index("s") * NC + lax.axis_index("c")
        base = wid * b_per_w
        half = F // 2
        tbl = table_hbm.at[0]
        pltpu.sync_copy(xT_hbm.at[:, pl.ds(base, b_per_w)], idx_v)

        def fire(f, grp_sem):
            pltpu.async_copy(tbl.at[idx_v.at[f]], rows_v.at[f], grp_sem)

        lax.fori_loop(0, half, lambda f, _: (fire(f, sem), 0)[1], 0)
        bias_cp = pltpu.async_copy(bias_hbm, bias_v.at[pl.ds(0, 1)], csem)
        lax.fori_loop(half, F, lambda f, _: (fire(f, bsem), 0)[1], 0)

        n_ch = b_per_w // 16

        def drain(grp_sem, n):
            def body(_, carry):
                pltpu.make_async_copy(
                    tbl.at[idx_v.at[0]], rows_v.at[0], grp_sem
                ).wait()
                return carry

            lax.fori_loop(0, n, body, 0)

        def accumulate(lo, hi, accs):
            def body(f, accs):
                return tuple(
                    accs[c] + rows_v[f, pl.ds(c * 16, 16)]
                    for c in range(n_ch)
                )

            return lax.fori_loop(lo, hi, body, accs)

        zeros = jnp.zeros((16,), jnp.float32)
        drain(sem, half)
        accs = accumulate(0, half, (zeros,) * n_ch)
        bias_cp.wait()
        drain(bsem, F - half)
        accs = accumulate(half, F, accs)
        b = bias_v[...][0]
        for c in range(n_ch):
            out_v[pl.ds(c * 16, 16)] = accs[c] + b
        pltpu.sync_copy(out_v, out_hbm.at[pl.ds(base, b_per_w)])

    return sc_k


def kernel(x, fc_weight, bias):
    B, F = x.shape
    info = plsc.get_sparse_core_info()
    NC, NS = info.num_cores, info.num_subcores
    NW = NC * NS
    b_per_w = B // NW

    xT = x.astype(jnp.int32).T
    table = fc_weight.reshape(1, -1)

    sc_k = _make_sc_kernel(B, F, b_per_w, NC)
    out = sc_k(xT, table, bias)
    return out.reshape(B, 1)
